# baseline (speedup 1.0000x reference)
import functools

import jax
import jax.numpy as jnp
from jax import lax
from jax.experimental import pallas as pl
from jax.experimental.pallas import tpu as pltpu
from jax.experimental.pallas import tpu_sc as plsc

NODE_NUM = 10000
NHID = 128
E_SUB = 320000
LANES = 16

NC = 2
NS = 16
NW = NC * NS
EPW = E_SUB // NW
CHUNK = 40
EPW_PAD = 10000
NPAIR = EPW_PAD // (2 * CHUNK)
NODE_PAD = 10112
ROWS_PER_TILE = NODE_PAD // NS


def _sc_aggregate(edge_emb, ph, idx_r, zeros):
    mesh = plsc.VectorSubcoreMesh(core_axis_name="c", subcore_axis_name="s")

    @functools.partial(
        pl.kernel,
        out_type=jax.ShapeDtypeStruct((NC, NODE_PAD, NHID), jnp.float32),
        mesh=mesh,
        scratch_types=[
            pltpu.VMEM((6, CHUNK), jnp.int32),
            pltpu.VMEM((6, CHUNK), jnp.int32),
            pltpu.VMEM((CHUNK, NHID), jnp.float32),
            pltpu.VMEM((CHUNK, NHID), jnp.float32),
            pltpu.VMEM((CHUNK, NHID), jnp.float32),
            pltpu.VMEM((CHUNK, NHID), jnp.float32),
            pltpu.VMEM((CHUNK, NHID), jnp.float32),
            pltpu.VMEM((CHUNK, NHID), jnp.float32),
            pltpu.VMEM((CHUNK,), jnp.int32),
            pltpu.VMEM_SHARED((NODE_PAD, NHID), jnp.float32),
            pltpu.SemaphoreType.DMA,
            pltpu.SemaphoreType.DMA,
            pltpu.SemaphoreType.DMA,
            pltpu.SemaphoreType.DMA,
            pltpu.SemaphoreType.DMA,
            pltpu.SemaphoreType.DMA,
            pltpu.SemaphoreType.DMA,
        ],
    )
    def k(edge_emb_h, ph_h, idx_h, zeros_h, out_h,
          i0, i1, ef0, ef1, nf0, nf1, mg0, mg1, ti, acc,
          se0, se1, sn0, sn1, ss0, ss1, si):
        c = lax.axis_index("c")
        s = lax.axis_index("s")
        wid = s * NC + c

        efs = (ef0, ef1)
        nfs = (nf0, nf1)
        mgs = (mg0, mg1)
        ses = (se0, se1)
        sns = (sn0, sn1)
        sss = (ss0, ss1)

        def issue(I, b):
            pltpu.async_copy(edge_emb_h.at[I.at[0 + b]], efs[b], ses[b])
            pltpu.async_copy(ph_h.at[I.at[2 + b]], nfs[b], sns[b])

        def wait(b):
            pltpu.make_async_copy(
                edge_emb_h.at[pl.ds(0, CHUNK)], efs[b], ses[b]).wait()
            pltpu.make_async_copy(
                ph_h.at[pl.ds(0, CHUNK)], nfs[b], sns[b]).wait()

        def wait_sc(b):
            pltpu.make_async_copy(
                edge_emb_h.at[pl.ds(0, CHUNK)], mgs[b], sss[b]).wait()

        def compute(b):
            ef = efs[b]
            nf = nfs[b]
            mg = mgs[b]

            def row_body(r, rc):
                r0 = 2 * r
                r1 = r0 + 1
                for j in range(NHID // LANES):
                    sl = pl.ds(j * LANES, LANES)
                    mg[r0, sl] = ef[r0, sl] * nf[r0, sl]
                for j in range(NHID // LANES):
                    sl = pl.ds(j * LANES, LANES)
                    mg[r1, sl] = ef[r1, sl] * nf[r1, sl]
                return rc

            lax.fori_loop(0, CHUNK // 2, row_body, 0)

        def scatter(I, b):
            pltpu.async_copy(mgs[b], acc.at[I.at[4 + b]], sss[b], add=True)

        def half(o, Icur, Inext):
            pltpu.async_copy(idx_h.at[wid, o + 1], Inext, si)
            for b in (0, 1):
                wait(b)
                wait_sc(b)
                compute(b)
                if b == 0:
                    pltpu.make_async_copy(
                        idx_h.at[wid, 0], Inext, si).wait()
                issue(Inext, b)
                scatter(Icur, b)

        iota = lax.iota(jnp.int32, LANES)
        ti[pl.ds(0, LANES)] = iota + NODE_NUM
        ti[pl.ds(LANES, LANES)] = iota + (NODE_NUM + LANES)
        ti[pl.ds(CHUNK - LANES, LANES)] = iota + (NODE_NUM + CHUNK - LANES)
        pltpu.async_copy(mg0, acc.at[ti], ss0, add=True)
        pltpu.async_copy(mg1, acc.at[ti], ss1, add=True)

        rows = pl.ds(s * ROWS_PER_TILE, ROWS_PER_TILE)
        pltpu.sync_copy(zeros_h.at[rows], acc.at[rows])
        plsc.subcore_barrier()

        pltpu.sync_copy(idx_h.at[wid, 0], i0)
        issue(i0, 0)
        issue(i0, 1)

        def outer(oo, carry):
            half(2 * oo, i0, i1)
            half(2 * oo + 1, i1, i0)
            return carry

        lax.fori_loop(0, (NPAIR - 1) // 2, outer, 0)

        for b in (0, 1):
            wait(b)
            wait_sc(b)
            compute(b)
            scatter(i0, b)

        wait_sc(0)
        wait_sc(1)
        plsc.subcore_barrier()
        pltpu.sync_copy(acc.at[rows], out_h.at[c, rows])

    return k(edge_emb, ph, idx_r, zeros)


BLK = 1000


def _tc_update(acc2, ph, W, b2):
    def body(a_ref, ph_ref, w_ref, b_ref, o_ref):
        x = a_ref[0] + a_ref[1]
        h = jnp.dot(x, w_ref[...], preferred_element_type=jnp.float32)
        h = jnp.maximum(h + b_ref[...], 0.0)
        o_ref[...] = h + ph_ref[...]

    return pl.pallas_call(
        body,
        grid=(NODE_NUM // BLK,),
        in_specs=[
            pl.BlockSpec((NC, BLK, NHID), lambda i: (0, i, 0)),
            pl.BlockSpec((BLK, NHID), lambda i: (i, 0)),
            pl.BlockSpec((NHID, NHID), lambda i: (0, 0)),
            pl.BlockSpec((1, NHID), lambda i: (0, 0)),
        ],
        out_specs=pl.BlockSpec((BLK, NHID), lambda i: (i, 0)),
        out_shape=jax.ShapeDtypeStruct((NODE_NUM, NHID), jnp.float32),
    )(acc2, ph, W, b2)


def _pad_edges(x, fill=None):
    if fill is None:
        pad = NODE_NUM + (jnp.arange(EPW_PAD - EPW, dtype=jnp.int32)
                          % (NODE_PAD - NODE_NUM - 8))
        pad = jnp.broadcast_to(pad, (NW, EPW_PAD - EPW))
        return jnp.concatenate([x.reshape(NW, EPW), pad],
                               axis=1).reshape(NW, NPAIR, 2, CHUNK)
    return jnp.pad(x.reshape(NW, EPW), ((0, 0), (0, EPW_PAD - EPW)),
                   constant_values=fill).reshape(NW, NPAIR, 2, CHUNK)


def kernel(projection_head, sub_edge, edge2index, edge_embedding, W, b):
    e2i = _pad_edges(edge2index, 0)
    src = _pad_edges(sub_edge[0], 0)
    dst = _pad_edges(sub_edge[1])
    idx_r = jnp.concatenate([e2i, src, dst], axis=2)
    zeros = jnp.zeros((NODE_PAD, NHID), jnp.float32)
    acc2 = _sc_aggregate(edge_embedding, projection_head, idx_r, zeros)
    return _tc_update(acc2, projection_head, W, b.reshape(1, NHID))


# speedup vs baseline: 1.0017x; 1.0017x over previous
import functools

import jax
import jax.numpy as jnp
from jax import lax
from jax.experimental import pallas as pl
from jax.experimental.pallas import tpu as pltpu
from jax.experimental.pallas import tpu_sc as plsc

NODE_NUM = 10000
NHID = 128
E_SUB = 320000
LANES = 16

NC = 2
NS = 16
NW = NC * NS
EPW = E_SUB // NW
CHUNK = 40
EPW_PAD = 10000
NPAIR = EPW_PAD // (2 * CHUNK)
NODE_PAD = 10112
ROWS_PER_TILE = NODE_PAD // NS


def _sc_aggregate(edge_emb, ph, idx_r, zeros):
    mesh = plsc.VectorSubcoreMesh(core_axis_name="c", subcore_axis_name="s")

    @functools.partial(
        pl.kernel,
        out_type=jax.ShapeDtypeStruct((NC, NODE_PAD, NHID), jnp.float32),
        mesh=mesh,
        scratch_types=[
            pltpu.VMEM((6, CHUNK), jnp.int32),
            pltpu.VMEM((6, CHUNK), jnp.int32),
            pltpu.VMEM((CHUNK, NHID), jnp.float32),
            pltpu.VMEM((CHUNK, NHID), jnp.float32),
            pltpu.VMEM((CHUNK, NHID), jnp.float32),
            pltpu.VMEM((CHUNK, NHID), jnp.float32),
            pltpu.VMEM((CHUNK, NHID), jnp.float32),
            pltpu.VMEM((CHUNK, NHID), jnp.float32),
            pltpu.VMEM((CHUNK,), jnp.int32),
            pltpu.VMEM_SHARED((NODE_PAD, NHID), jnp.float32),
            pltpu.SemaphoreType.DMA,
            pltpu.SemaphoreType.DMA,
            pltpu.SemaphoreType.DMA,
            pltpu.SemaphoreType.DMA,
            pltpu.SemaphoreType.DMA,
            pltpu.SemaphoreType.DMA,
            pltpu.SemaphoreType.DMA,
        ],
    )
    def k(edge_emb_h, ph_h, idx_h, zeros_h, out_h,
          i0, i1, ef0, ef1, nf0, nf1, mg0, mg1, ti, acc,
          se0, se1, sn0, sn1, ss0, ss1, si):
        c = lax.axis_index("c")
        s = lax.axis_index("s")
        wid = s * NC + c

        efs = (ef0, ef1)
        nfs = (nf0, nf1)
        mgs = (mg0, mg1)
        ses = (se0, se1)
        sns = (sn0, sn1)
        sss = (ss0, ss1)

        def issue(I, b):
            pltpu.async_copy(edge_emb_h.at[I.at[0 + b]], efs[b], ses[b])
            pltpu.async_copy(ph_h.at[I.at[2 + b]], nfs[b], sns[b])

        def wait(b):
            pltpu.make_async_copy(
                edge_emb_h.at[pl.ds(0, CHUNK)], efs[b], ses[b]).wait()
            pltpu.make_async_copy(
                ph_h.at[pl.ds(0, CHUNK)], nfs[b], sns[b]).wait()

        def wait_sc(b):
            pltpu.make_async_copy(
                edge_emb_h.at[pl.ds(0, CHUNK)], mgs[b], sss[b]).wait()

        def compute(b):
            ef = efs[b]
            nf = nfs[b]
            mg = mgs[b]

            def row_body(r, rc):
                for j in range(NHID // LANES):
                    sl = pl.ds(j * LANES, LANES)
                    mg[r, sl] = ef[r, sl] * nf[r, sl]
                return rc

            lax.fori_loop(0, CHUNK, row_body, 0)

        def scatter(I, b):
            pltpu.async_copy(mgs[b], acc.at[I.at[4 + b]], sss[b], add=True)

        def half(o, Icur, Inext):
            pltpu.async_copy(idx_h.at[wid, o + 1], Inext, si)
            for b in (0, 1):
                wait(b)
                wait_sc(b)
                compute(b)
                if b == 0:
                    pltpu.make_async_copy(
                        idx_h.at[wid, 0], Inext, si).wait()
                issue(Inext, b)
                scatter(Icur, b)

        iota = lax.iota(jnp.int32, LANES)
        ti[pl.ds(0, LANES)] = iota + NODE_NUM
        ti[pl.ds(LANES, LANES)] = iota + (NODE_NUM + LANES)
        ti[pl.ds(CHUNK - LANES, LANES)] = iota + (NODE_NUM + CHUNK - LANES)
        pltpu.async_copy(mg0, acc.at[ti], ss0, add=True)
        pltpu.async_copy(mg1, acc.at[ti], ss1, add=True)

        rows = pl.ds(s * ROWS_PER_TILE, ROWS_PER_TILE)
        pltpu.sync_copy(zeros_h.at[rows], acc.at[rows])
        plsc.subcore_barrier()

        pltpu.sync_copy(idx_h.at[wid, 0], i0)
        issue(i0, 0)
        issue(i0, 1)

        def outer(oo, carry):
            half(2 * oo, i0, i1)
            half(2 * oo + 1, i1, i0)
            return carry

        lax.fori_loop(0, (NPAIR - 1) // 2, outer, 0)

        for b in (0, 1):
            wait(b)
            wait_sc(b)
            compute(b)
            scatter(i0, b)

        wait_sc(0)
        wait_sc(1)
        plsc.subcore_barrier()
        pltpu.sync_copy(acc.at[rows], out_h.at[c, rows])

    return k(edge_emb, ph, idx_r, zeros)


BLK = 1000


def _tc_update(acc2, ph, W, b2):
    def body(a_ref, ph_ref, w_ref, b_ref, o_ref):
        x = a_ref[0] + a_ref[1]
        h = jnp.dot(x, w_ref[...], preferred_element_type=jnp.float32)
        h = jnp.maximum(h + b_ref[...], 0.0)
        o_ref[...] = h + ph_ref[...]

    return pl.pallas_call(
        body,
        grid=(NODE_NUM // BLK,),
        in_specs=[
            pl.BlockSpec((NC, BLK, NHID), lambda i: (0, i, 0)),
            pl.BlockSpec((BLK, NHID), lambda i: (i, 0)),
            pl.BlockSpec((NHID, NHID), lambda i: (0, 0)),
            pl.BlockSpec((1, NHID), lambda i: (0, 0)),
        ],
        out_specs=pl.BlockSpec((BLK, NHID), lambda i: (i, 0)),
        out_shape=jax.ShapeDtypeStruct((NODE_NUM, NHID), jnp.float32),
    )(acc2, ph, W, b2)


def _split_edges(x):
    return x.reshape(NW, NPAIR, 2, CHUNK)


def kernel(projection_head, sub_edge, edge2index, edge_embedding, W, b):
    e2i = _split_edges(edge2index)
    src = _split_edges(sub_edge[0])
    dst = _split_edges(sub_edge[1])
    idx_r = jnp.concatenate([e2i, src, dst], axis=2)
    zeros = jnp.zeros((NODE_PAD, NHID), jnp.float32)
    acc2 = _sc_aggregate(edge_embedding, projection_head, idx_r, zeros)
    return _tc_update(acc2, projection_head, W, b.reshape(1, NHID))


# speedup vs baseline: 1.0173x; 1.0155x over previous
import functools

import jax
import jax.numpy as jnp
from jax import lax
from jax.experimental import pallas as pl
from jax.experimental.pallas import tpu as pltpu
from jax.experimental.pallas import tpu_sc as plsc

NODE_NUM = 10000
NHID = 128
E_SUB = 320000
LANES = 16

NC = 2
NS = 16
NW = NC * NS
EPW = E_SUB // NW
CHUNK = 40
EPW_PAD = 10000
NPAIR = EPW_PAD // (2 * CHUNK)
NODE_PAD = 10112
ROWS_PER_TILE = NODE_PAD // NS


def _sc_aggregate(edge_emb, ph, idx_r):
    mesh = plsc.VectorSubcoreMesh(core_axis_name="c", subcore_axis_name="s")

    @functools.partial(
        pl.kernel,
        out_type=jax.ShapeDtypeStruct((NC, NODE_PAD, NHID), jnp.float32),
        mesh=mesh,
        scratch_types=[
            pltpu.VMEM((6, CHUNK), jnp.int32),
            pltpu.VMEM((6, CHUNK), jnp.int32),
            pltpu.VMEM((CHUNK, NHID), jnp.float32),
            pltpu.VMEM((CHUNK, NHID), jnp.float32),
            pltpu.VMEM((CHUNK, NHID), jnp.float32),
            pltpu.VMEM((CHUNK, NHID), jnp.float32),
            pltpu.VMEM((CHUNK, NHID), jnp.float32),
            pltpu.VMEM((CHUNK, NHID), jnp.float32),
            pltpu.VMEM((CHUNK,), jnp.int32),
            pltpu.VMEM_SHARED((NODE_PAD, NHID), jnp.float32),
            pltpu.SemaphoreType.DMA,
            pltpu.SemaphoreType.DMA,
            pltpu.SemaphoreType.DMA,
            pltpu.SemaphoreType.DMA,
            pltpu.SemaphoreType.DMA,
            pltpu.SemaphoreType.DMA,
            pltpu.SemaphoreType.DMA,
        ],
    )
    def k(edge_emb_h, ph_h, idx_h, out_h,
          i0, i1, ef0, ef1, nf0, nf1, mg0, mg1, ti, acc,
          se0, se1, sn0, sn1, ss0, ss1, si):
        c = lax.axis_index("c")
        s = lax.axis_index("s")
        wid = s * NC + c

        efs = (ef0, ef1)
        nfs = (nf0, nf1)
        mgs = (mg0, mg1)
        ses = (se0, se1)
        sns = (sn0, sn1)
        sss = (ss0, ss1)

        def issue(I, b):
            pltpu.async_copy(edge_emb_h.at[I.at[0 + b]], efs[b], ses[b])
            pltpu.async_copy(ph_h.at[I.at[2 + b]], nfs[b], sns[b])

        def wait(b):
            pltpu.make_async_copy(
                edge_emb_h.at[pl.ds(0, CHUNK)], efs[b], ses[b]).wait()
            pltpu.make_async_copy(
                ph_h.at[pl.ds(0, CHUNK)], nfs[b], sns[b]).wait()

        def wait_sc(b):
            pltpu.make_async_copy(
                edge_emb_h.at[pl.ds(0, CHUNK)], mgs[b], sss[b]).wait()

        def compute(b):
            ef = efs[b]
            nf = nfs[b]
            mg = mgs[b]

            def row_body(r, rc):
                for j in range(NHID // LANES):
                    sl = pl.ds(j * LANES, LANES)
                    mg[r, sl] = ef[r, sl] * nf[r, sl]
                return rc

            lax.fori_loop(0, CHUNK, row_body, 0)

        def scatter(I, b):
            pltpu.async_copy(mgs[b], acc.at[I.at[4 + b]], sss[b], add=True)

        def half(o, Icur, Inext):
            pltpu.async_copy(idx_h.at[wid, o + 1], Inext, si)
            for b in (0, 1):
                wait(b)
                wait_sc(b)
                compute(b)
                if b == 0:
                    pltpu.make_async_copy(
                        idx_h.at[wid, 0], Inext, si).wait()
                issue(Inext, b)
                scatter(Icur, b)

        iota = lax.iota(jnp.int32, LANES)
        ti[pl.ds(0, LANES)] = iota + NODE_NUM
        ti[pl.ds(LANES, LANES)] = iota + (NODE_NUM + LANES)
        ti[pl.ds(CHUNK - LANES, LANES)] = iota + (NODE_NUM + CHUNK - LANES)
        pltpu.async_copy(mg0, acc.at[ti], ss0, add=True)
        pltpu.async_copy(mg1, acc.at[ti], ss1, add=True)

        zf = jnp.zeros((LANES,), jnp.float32)

        def zrow(r, rc):
            for j in range(NHID // LANES):
                mg0[r, pl.ds(j * LANES, LANES)] = zf
            return rc

        lax.fori_loop(0, CHUNK, zrow, 0)
        base = s * ROWS_PER_TILE

        def zcopy(kk, rc):
            pltpu.sync_copy(mg0, acc.at[pl.ds(base + kk * CHUNK, CHUNK)])
            return rc

        lax.fori_loop(0, ROWS_PER_TILE // CHUNK, zcopy, 0)
        tail = ROWS_PER_TILE % CHUNK
        pltpu.sync_copy(mg0.at[pl.ds(0, tail)],
                        acc.at[pl.ds(base + ROWS_PER_TILE - tail, tail)])
        rows = pl.ds(base, ROWS_PER_TILE)
        plsc.subcore_barrier()

        pltpu.sync_copy(idx_h.at[wid, 0], i0)
        issue(i0, 0)
        issue(i0, 1)

        def outer(oo, carry):
            half(2 * oo, i0, i1)
            half(2 * oo + 1, i1, i0)
            return carry

        lax.fori_loop(0, (NPAIR - 1) // 2, outer, 0)

        for b in (0, 1):
            wait(b)
            wait_sc(b)
            compute(b)
            scatter(i0, b)

        wait_sc(0)
        wait_sc(1)
        plsc.subcore_barrier()
        pltpu.sync_copy(acc.at[rows], out_h.at[c, rows])

    return k(edge_emb, ph, idx_r)


BLK = 1000


def _tc_update(acc2, ph, W, b2):
    def body(a_ref, ph_ref, w_ref, b_ref, o_ref):
        x = a_ref[0] + a_ref[1]
        h = jnp.dot(x, w_ref[...], preferred_element_type=jnp.float32)
        h = jnp.maximum(h + b_ref[...], 0.0)
        o_ref[...] = h + ph_ref[...]

    return pl.pallas_call(
        body,
        grid=(NODE_NUM // BLK,),
        in_specs=[
            pl.BlockSpec((NC, BLK, NHID), lambda i: (0, i, 0)),
            pl.BlockSpec((BLK, NHID), lambda i: (i, 0)),
            pl.BlockSpec((NHID, NHID), lambda i: (0, 0)),
            pl.BlockSpec((1, NHID), lambda i: (0, 0)),
        ],
        out_specs=pl.BlockSpec((BLK, NHID), lambda i: (i, 0)),
        out_shape=jax.ShapeDtypeStruct((NODE_NUM, NHID), jnp.float32),
    )(acc2, ph, W, b2)


def _split_edges(x):
    return x.reshape(NW, NPAIR, 2, CHUNK)


def kernel(projection_head, sub_edge, edge2index, edge_embedding, W, b):
    e2i = _split_edges(edge2index)
    src = _split_edges(sub_edge[0])
    dst = _split_edges(sub_edge[1])
    idx_r = jnp.concatenate([e2i, src, dst], axis=2)
    acc2 = _sc_aggregate(edge_embedding, projection_head, idx_r)
    return _tc_update(acc2, projection_head, W, b.reshape(1, NHID))
